# baseline (speedup 1.0000x reference)
import functools

import jax
import jax.numpy as jnp
from jax import lax
from jax.experimental import pallas as pl
from jax.experimental.pallas import tpu as pltpu
from jax.experimental.pallas import tpu_sc as plsc

S_BLK = 125
NQ = 1024
NS = 4000
NM = 9
NT = 128
C3N = NM * 64


def _cg_densify(idx_pad, cg_pad):
    mesh = plsc.VectorSubcoreMesh(core_axis_name="c", subcore_axis_name="s")

    @functools.partial(
        pl.kernel, mesh=mesh,
        out_type=jax.ShapeDtypeStruct((C3N,), jnp.float32),
        compiler_params=pltpu.CompilerParams(needs_layout_passes=False),
        scratch_types=[
            pltpu.VMEM((NT,), jnp.int32),
            pltpu.VMEM((NT,), jnp.float32),
            pltpu.VMEM((C3N,), jnp.float32),
        ],
    )
    def _k(idx_hbm, cg_hbm, out_hbm, idx_v, cg_v, c_v):
        @pl.when((lax.axis_index("c") == 0) & (lax.axis_index("s") == 0))
        def _():
            pltpu.sync_copy(idx_hbm, idx_v)
            pltpu.sync_copy(cg_hbm, cg_v)
            for i in range(C3N // 16):
                c_v[pl.ds(i * 16, 16)] = jnp.zeros((16,), jnp.float32)
            for i in range(NT // 16):
                sl = pl.ds(i * 16, 16)
                plsc.addupdate_scatter(c_v, [idx_v[sl]], cg_v[sl])
            pltpu.sync_copy(c_v, out_hbm)

    return _k(idx_pad, cg_pad)


def _combine_body(c3_ref, nu_ref, b1_ref, sel0_ref, sel1_ref, out_ref,
                  oh0_ref, oh1_ref):
    @pl.when(pl.program_id(0) == 0)
    def _init():
        i0 = lax.broadcasted_iota(jnp.int32, (256, NQ), 0)
        oh0_ref[...] = (i0 == sel0_ref[...]).astype(jnp.bfloat16)
        i1 = lax.broadcasted_iota(jnp.int32, (128, NQ), 0)
        oh1_ref[...] = (i1 == sel1_ref[...]).astype(jnp.bfloat16)

    z256 = jnp.zeros((S_BLK, 1, 256), jnp.float32)
    z128 = jnp.zeros((S_BLK, 1, 128), jnp.float32)
    nu2 = jnp.concatenate([nu_ref[...], z256], axis=1).reshape(
        S_BLK * 8, 256).astype(jnp.bfloat16)
    b12 = jnp.concatenate([b1_ref[...], z128], axis=1).reshape(
        S_BLK * 8, 128).astype(jnp.bfloat16)
    c3 = c3_ref[...].astype(jnp.bfloat16)
    a = jnp.dot(nu2, oh0_ref[...], preferred_element_type=jnp.float32)
    b = jnp.dot(b12, oh1_ref[...], preferred_element_type=jnp.float32)
    a16 = a.astype(jnp.bfloat16).reshape(S_BLK, 8, NQ)
    b16 = b.astype(jnp.bfloat16).reshape(S_BLK, 8, NQ)
    for s in range(S_BLK):
        asx = a16[s]
        bsy = b16[s]
        p = (asx[:, None, :] * bsy[None, :, :]).reshape(64, NQ)
        out_ref[s] = jnp.dot(c3, p, preferred_element_type=jnp.float32)


def _combine(c3, nu_p, b1_p, sel0, sel1, interpret=False):
    ns = nu_p.shape[0]
    return pl.pallas_call(
        _combine_body,
        grid=(ns // S_BLK,),
        in_specs=[
            pl.BlockSpec((NM, 64), lambda i: (0, 0)),
            pl.BlockSpec((S_BLK, 7, 256), lambda i: (i, 0, 0)),
            pl.BlockSpec((S_BLK, 7, 128), lambda i: (i, 0, 0)),
            pl.BlockSpec((1, NQ), lambda i: (0, 0)),
            pl.BlockSpec((1, NQ), lambda i: (0, 0)),
        ],
        out_specs=pl.BlockSpec((S_BLK, NM, NQ), lambda i: (i, 0, 0)),
        out_shape=jax.ShapeDtypeStruct((ns, NM, NQ), jnp.float32),
        scratch_shapes=[
            pltpu.VMEM((256, NQ), jnp.bfloat16),
            pltpu.VMEM((128, NQ), jnp.bfloat16),
        ],
        interpret=interpret,
    )(c3, nu_p, b1_p, sel0, sel1)


def kernel(block_nu_values, block_1_values, selected_features,
           mu_array, m_array, M_array, cg_array):
    sel0 = selected_features[:, 0].astype(jnp.int32).reshape(1, NQ)
    sel1 = selected_features[:, 1].astype(jnp.int32).reshape(1, NQ)
    nt = mu_array.shape[0]
    idx = (M_array.astype(jnp.int32) * 64 + mu_array.astype(jnp.int32) * 8
           + m_array.astype(jnp.int32))
    idx_pad = jnp.concatenate([idx, jnp.full((NT - nt,), 7, jnp.int32)])
    cg_pad = jnp.concatenate(
        [cg_array.astype(jnp.float32), jnp.zeros((NT - nt,), jnp.float32)])
    def _stage(idxp, cgp, nu, b1, s0, s1):
        c3 = _cg_densify(idxp, cgp).reshape(NM, 64)
        return _combine(c3, nu, b1, s0, s1)

    devs = jax.devices()
    nd = 2 if len(devs) >= 2 else 1
    ns = block_nu_values.shape[0]
    if nd > 1 and ns % (nd * S_BLK) == 0:
        mesh = jax.make_mesh((nd,), ("d",), devices=devs[:nd])
        ps = jax.sharding.PartitionSpec
        nsh = lambda spec: jax.sharding.NamedSharding(mesh, spec)
        args = (
            jax.reshard(idx_pad, nsh(ps())),
            jax.reshard(cg_pad, nsh(ps())),
            jax.reshard(block_nu_values, nsh(ps("d"))),
            jax.reshard(block_1_values, nsh(ps("d"))),
            jax.reshard(sel0, nsh(ps())),
            jax.reshard(sel1, nsh(ps())),
        )
        return jax.shard_map(
            _stage, mesh=mesh,
            in_specs=(ps(), ps(), ps("d"), ps("d"), ps(), ps()),
            out_specs=ps("d"), check_vma=False,
        )(*args)
    return _stage(idx_pad, cg_pad, block_nu_values, block_1_values, sel0, sel1)


# speedup vs baseline: 1.7861x; 1.7861x over previous
import functools

import jax
import jax.numpy as jnp
from jax import lax
from jax.experimental import pallas as pl
from jax.experimental.pallas import tpu as pltpu
from jax.experimental.pallas import tpu_sc as plsc

S_BLK = 160
NQ = 1024
NS = 4000
NM = 9
NT = 128
C3N = NM * 64


def _cg_densify(idx_pad, cg_pad):
    mesh = plsc.VectorSubcoreMesh(core_axis_name="c", subcore_axis_name="s")

    @functools.partial(
        pl.kernel, mesh=mesh,
        out_type=jax.ShapeDtypeStruct((C3N,), jnp.float32),
        compiler_params=pltpu.CompilerParams(needs_layout_passes=False),
        scratch_types=[
            pltpu.VMEM((NT,), jnp.int32),
            pltpu.VMEM((NT,), jnp.float32),
            pltpu.VMEM((C3N,), jnp.float32),
        ],
    )
    def _k(idx_hbm, cg_hbm, out_hbm, idx_v, cg_v, c_v):
        @pl.when((lax.axis_index("c") == 0) & (lax.axis_index("s") == 0))
        def _():
            pltpu.sync_copy(idx_hbm, idx_v)
            pltpu.sync_copy(cg_hbm, cg_v)
            for i in range(C3N // 16):
                c_v[pl.ds(i * 16, 16)] = jnp.zeros((16,), jnp.float32)
            for i in range(NT // 16):
                sl = pl.ds(i * 16, 16)
                plsc.addupdate_scatter(c_v, [idx_v[sl]], cg_v[sl])
            pltpu.sync_copy(c_v, out_hbm)

    return _k(idx_pad, cg_pad)


def _combine_body(c3_ref, nu_ref, b1_ref, sel0_ref, sel1_ref, out_ref,
                  oh0_ref, oh1_ref):
    @pl.when(pl.program_id(0) == 0)
    def _init():
        i0 = lax.broadcasted_iota(jnp.int32, (256, NQ), 0)
        oh0_ref[...] = (i0 == sel0_ref[...]).astype(jnp.bfloat16)
        i1 = lax.broadcasted_iota(jnp.int32, (128, NQ), 0)
        oh1_ref[...] = (i1 == sel1_ref[...]).astype(jnp.bfloat16)

    z256 = jnp.zeros((S_BLK, 1, 256), jnp.float32)
    z128 = jnp.zeros((S_BLK, 1, 128), jnp.float32)
    nu2 = jnp.concatenate([nu_ref[...], z256], axis=1).reshape(
        S_BLK * 8, 256).astype(jnp.bfloat16)
    b12 = jnp.concatenate([b1_ref[...], z128], axis=1).reshape(
        S_BLK * 8, 128).astype(jnp.bfloat16)
    c3 = c3_ref[...].astype(jnp.bfloat16)
    a = jnp.dot(nu2, oh0_ref[...], preferred_element_type=jnp.float32)
    b = jnp.dot(b12, oh1_ref[...], preferred_element_type=jnp.float32)
    a16 = a.astype(jnp.bfloat16).reshape(S_BLK, 8, NQ)
    b16 = b.astype(jnp.bfloat16).reshape(S_BLK, 8, NQ)
    for s in range(S_BLK):
        asx = a16[s]
        bsy = b16[s]
        p = (asx[:, None, :] * bsy[None, :, :]).reshape(64, NQ)
        out_ref[s] = jnp.dot(c3, p, preferred_element_type=jnp.float32)


def _combine(c3, nu_p, b1_p, sel0, sel1, interpret=False):
    return pl.pallas_call(
        _combine_body,
        grid=(NS // S_BLK,),
        in_specs=[
            pl.BlockSpec((NM, 64), lambda i: (0, 0)),
            pl.BlockSpec((S_BLK, 7, 256), lambda i: (i, 0, 0)),
            pl.BlockSpec((S_BLK, 7, 128), lambda i: (i, 0, 0)),
            pl.BlockSpec((1, NQ), lambda i: (0, 0)),
            pl.BlockSpec((1, NQ), lambda i: (0, 0)),
        ],
        out_specs=pl.BlockSpec((S_BLK, NM, NQ), lambda i: (i, 0, 0)),
        out_shape=jax.ShapeDtypeStruct((NS, NM, NQ), jnp.float32),
        scratch_shapes=[
            pltpu.VMEM((256, NQ), jnp.bfloat16),
            pltpu.VMEM((128, NQ), jnp.bfloat16),
        ],
        interpret=interpret,
    )(c3, nu_p, b1_p, sel0, sel1)


def kernel(block_nu_values, block_1_values, selected_features,
           mu_array, m_array, M_array, cg_array):
    sel0 = selected_features[:, 0].astype(jnp.int32).reshape(1, NQ)
    sel1 = selected_features[:, 1].astype(jnp.int32).reshape(1, NQ)
    nt = mu_array.shape[0]
    idx = (M_array.astype(jnp.int32) * 64 + mu_array.astype(jnp.int32) * 8
           + m_array.astype(jnp.int32))
    idx_pad = jnp.concatenate([idx, jnp.full((NT - nt,), 7, jnp.int32)])
    cg_pad = jnp.concatenate(
        [cg_array.astype(jnp.float32), jnp.zeros((NT - nt,), jnp.float32)])
    c3 = _cg_densify(idx_pad, cg_pad).reshape(NM, 64)
    return _combine(c3, block_nu_values, block_1_values, sel0, sel1)
